# baseline (speedup 1.0000x reference)
import functools
import jax
import jax.numpy as jnp
from jax import lax
from jax.experimental import pallas as pl
from jax.experimental.pallas import tpu as pltpu, tpu_sc as plsc

USERS = 1000000
F = 64
B = 16384
NC = 2
NS = 16
NW = NC * NS
BPW = B // NW
NCHUNK = 4
CW = BPW // NCHUNK
GROUPS = BPW // 16


def _gmf_body(idx_hbm, item_hbm, table_hbm, wb_hbm, out_hbm,
              idx_v, rows_v, item_v, wb_v, out_v, gsem):
    wid = lax.axis_index("s") * NC + lax.axis_index("c")
    base = wid * BPW

    pltpu.sync_copy(idx_hbm.at[wid], idx_v)
    copies = []
    for j in range(NCHUNK):
        copies.append(
            pltpu.async_copy(table_hbm.at[idx_v.at[j]],
                             rows_v.at[pl.ds(j * CW, CW)], gsem))
    pltpu.sync_copy(item_hbm.at[pl.ds(base, BPW)], item_v)
    pltpu.sync_copy(wb_hbm, wb_v)
    for c in copies:
        c.wait()

    bvec = plsc.load_gather(wb_v, [jnp.full((16,), F, jnp.int32)])

    def group(g, carry):
        row = g * 16 + lax.iota(jnp.int32, 16)
        acc = jnp.zeros((16,), jnp.float32)
        for f in range(F):
            fidx = jnp.full((16,), f, jnp.int32)
            uf = plsc.load_gather(rows_v, [row, fidx])
            itf = plsc.load_gather(item_v, [row, fidx])
            wf = plsc.load_gather(wb_v, [fidx])
            acc = acc + uf * itf * wf
        logit = acc + bvec
        out_v[pl.ds(g * 16, 16)] = 1.0 / (1.0 + jnp.exp(-logit))
        return carry

    lax.fori_loop(0, GROUPS, group, 0)
    pltpu.sync_copy(out_v, out_hbm.at[pl.ds(base, BPW)])


@functools.partial(jax.jit, static_argnames=())
def _gmf(user_idx_c, item_embedding, emb_table, wb):
    mesh = plsc.VectorSubcoreMesh(core_axis_name="c", subcore_axis_name="s")
    run = pl.kernel(
        _gmf_body,
        out_type=jax.ShapeDtypeStruct((B,), jnp.float32),
        mesh=mesh,
        compiler_params=pltpu.CompilerParams(
            needs_layout_passes=False, use_tc_tiling_on_sc=False),
        scratch_types=[
            pltpu.VMEM((NCHUNK, CW), jnp.int32),
            pltpu.VMEM((BPW, F), jnp.float32),
            pltpu.VMEM((BPW, F), jnp.float32),
            pltpu.VMEM((F + 16,), jnp.float32),
            pltpu.VMEM((BPW,), jnp.float32),
            pltpu.SemaphoreType.DMA,
        ],
    )
    return run(user_idx_c, item_embedding, emb_table, wb)


def kernel(user_idx, item_embedding, emb_table, fc_w, fc_b):
    idx = user_idx.astype(jnp.int32).reshape(NW, NCHUNK, CW)
    wb = jnp.concatenate(
        [fc_w.reshape(F), jnp.broadcast_to(fc_b.reshape(1), (16,))]
    ).astype(jnp.float32)
    return _gmf(idx, item_embedding, emb_table, wb)


# speedup vs baseline: 1.6887x; 1.6887x over previous
import functools
import jax
import jax.numpy as jnp
from jax import lax
from jax.experimental import pallas as pl
from jax.experimental.pallas import tpu as pltpu, tpu_sc as plsc

USERS = 1000000
F = 64
B = 16384
NC = 2
NS = 16
NW = NC * NS
BPW = B // NW
NWIN = (USERS + 127) // 128
WPW = 246
GROUPS = BPW // 16
SROWS = 128


def _p1_body(idx_hbm, tableT, gout,
             idx_v, kept_u, kept_i, ml_c, ml_i,
             blk_a, blk_b, staging, sidx,
             sem_a, sem_b, sem_s):
    wid = lax.axis_index("s") * NC + lax.axis_index("c")
    wlo = wid * WPW

    pltpu.sync_copy(idx_hbm, idx_v)

    lane = lax.iota(jnp.int32, 16)

    def fbody(j, kept):
        v = idx_v[pl.ds(j * 16, 16)]
        w = v >> 7
        pos = j * 16 + lane
        m = (w >= wlo) & (w < wlo + WPW)
        plsc.store_compressed(kept_u.at[pl.ds(kept, 16)], v, mask=m)
        plsc.store_compressed(kept_i.at[pl.ds(kept, 16)], pos, mask=m)
        return kept + jnp.sum(m.astype(jnp.int32))

    kept_n = lax.fori_loop(0, B // 16, fbody, 0)
    kchunks = (kept_n + 15) >> 4

    for t in range(8):
        sidx[0, pl.ds(t * 16, 16)] = jnp.full((16,), -1, jnp.int32)

    fv = [lane + 16 * t for t in range(4)]

    def fetch(w_abs, buf, sem):
        off = pl.multiple_of(jnp.minimum(w_abs, NWIN - 1) * 128, 128)
        pltpu.async_copy(tableT.at[:, pl.ds(off, 128)], buf, sem)

    def bwait(buf, sem):
        pltpu.make_async_copy(tableT.at[:, pl.ds(0, 128)], buf, sem).wait()

    def flush():
        pltpu.async_copy(
            staging,
            gout.at[plsc.Indices(sidx.at[0], ignored_value=-1)],
            sem_s).wait()
        for t in range(8):
            sidx[0, pl.ds(t * 16, 16)] = jnp.full((16,), -1, jnp.int32)

    def process(buf, w_abs, sc_in):
        def mbody(j, mc):
            u = kept_u[pl.ds(j * 16, 16)]
            pos = j * 16 + lane
            m = ((u >> 7) == w_abs) & (pos < kept_n)
            plsc.store_compressed(ml_c.at[pl.ds(mc, 16)], u & 127, mask=m)
            i = kept_i[pl.ds(j * 16, 16)]
            plsc.store_compressed(ml_i.at[pl.ds(mc, 16)], i, mask=m)
            return mc + jnp.sum(m.astype(jnp.int32))

        mcount = lax.fori_loop(0, kchunks, mbody, 0)
        mchunks = (mcount + 15) >> 4

        def cbody(jj, sc):
            cvec = ml_c[pl.ds(jj * 16, 16)] & 127
            ivec = ml_i[pl.ds(jj * 16, 16)]
            valid = (jj * 16 + lane) < mcount
            srow = sc & (SROWS - 1)
            sidx[0, pl.ds(srow, 16)] = jnp.where(valid, ivec, -1)
            for k2 in range(16):
                cv = jnp.full((16,), 0, jnp.int32) + cvec[k2]
                for t in range(4):
                    vec = plsc.load_gather(buf, [fv[t], cv])
                    staging[srow + k2, pl.ds(t * 16, 16)] = vec
            sc1 = sc + 16

            @pl.when((sc1 & (SROWS - 1)) == 0)
            def _():
                flush()

            return sc1

        return lax.fori_loop(0, mchunks, cbody, sc_in)

    fetch(wlo, blk_a, sem_a)

    def wbody(g, sc):
        fetch(wlo + 2 * g + 1, blk_b, sem_b)
        bwait(blk_a, sem_a)
        sc = process(blk_a, wlo + 2 * g, sc)
        fetch(wlo + 2 * g + 2, blk_a, sem_a)
        bwait(blk_b, sem_b)
        sc = process(blk_b, wlo + 2 * g + 1, sc)
        return sc

    sc = lax.fori_loop(0, WPW // 2, wbody, 0)
    bwait(blk_a, sem_a)

    @pl.when((sc & (SROWS - 1)) != 0)
    def _():
        flush()


def _p2_body(g_hbm, item_hbm, wb_hbm, out_hbm,
             g_v, item_v, wb_v, out_v, sem):
    wid = lax.axis_index("s") * NC + lax.axis_index("c")
    base = wid * BPW

    copy = pltpu.async_copy(g_hbm.at[pl.ds(base, BPW), :], g_v, sem)
    pltpu.sync_copy(item_hbm.at[pl.ds(base, BPW)], item_v)
    pltpu.sync_copy(wb_hbm, wb_v)
    copy.wait()

    bvec = plsc.load_gather(wb_v, [jnp.full((16,), F, jnp.int32)])
    lane = lax.iota(jnp.int32, 16)

    def group(g, carry):
        row = g * 16 + lane
        acc = jnp.zeros((16,), jnp.float32)
        for f in range(F):
            fidx = jnp.full((16,), f, jnp.int32)
            uf = plsc.load_gather(g_v, [row, fidx])
            itf = plsc.load_gather(item_v, [row, fidx])
            wf = plsc.load_gather(wb_v, [fidx])
            acc = acc + uf * itf * wf
        logit = acc + bvec
        out_v[pl.ds(g * 16, 16)] = 1.0 / (1.0 + jnp.exp(-logit))
        return carry

    lax.fori_loop(0, GROUPS, group, 0)
    pltpu.sync_copy(out_v, out_hbm.at[pl.ds(base, BPW)])


@jax.jit
def _gmf(user_idx_c, item_embedding, emb_table, wb):
    mesh = plsc.VectorSubcoreMesh(core_axis_name="c", subcore_axis_name="s")
    tableT = emb_table.T

    gathered = pl.kernel(
        _p1_body,
        out_type=jax.ShapeDtypeStruct((B, 128), jnp.float32),
        mesh=mesh,
        compiler_params=pltpu.CompilerParams(
            needs_layout_passes=False, use_tc_tiling_on_sc=True),
        scratch_types=[
            pltpu.VMEM((B,), jnp.int32),
            pltpu.VMEM((B,), jnp.int32),
            pltpu.VMEM((B,), jnp.int32),
            pltpu.VMEM((B,), jnp.int32),
            pltpu.VMEM((B,), jnp.int32),
            pltpu.VMEM((F, 128), jnp.float32),
            pltpu.VMEM((F, 128), jnp.float32),
            pltpu.VMEM((SROWS, 128), jnp.float32),
            pltpu.VMEM((1, SROWS), jnp.int32),
            pltpu.SemaphoreType.DMA,
            pltpu.SemaphoreType.DMA,
            pltpu.SemaphoreType.DMA,
        ],
    )(user_idx_c, tableT)

    out = pl.kernel(
        _p2_body,
        out_type=jax.ShapeDtypeStruct((B,), jnp.float32),
        mesh=mesh,
        compiler_params=pltpu.CompilerParams(
            needs_layout_passes=False, use_tc_tiling_on_sc=False),
        scratch_types=[
            pltpu.VMEM((BPW, 128), jnp.float32),
            pltpu.VMEM((BPW, F), jnp.float32),
            pltpu.VMEM((F + 16,), jnp.float32),
            pltpu.VMEM((BPW,), jnp.float32),
            pltpu.SemaphoreType.DMA,
        ],
    )(gathered, item_embedding, wb)
    return out


def kernel(user_idx, item_embedding, emb_table, fc_w, fc_b):
    idx = user_idx.astype(jnp.int32)
    wb = jnp.concatenate(
        [fc_w.reshape(F), jnp.broadcast_to(fc_b.reshape(1), (16,))]
    ).astype(jnp.float32)
    return _gmf(idx, item_embedding, emb_table, wb)


# speedup vs baseline: 1.7370x; 1.0286x over previous
import functools
import jax
import jax.numpy as jnp
from jax import lax
from jax.experimental import pallas as pl
from jax.experimental.pallas import tpu as pltpu, tpu_sc as plsc

USERS = 1000000
F = 64
B = 16384
NC = 2
NS = 16
NW = NC * NS
BPW = B // NW
NWIN = (USERS + 127) // 128
NBUF = 8
WPW = 248
GROUPS = BPW // 16
SROWS = 64


def _p1_body(idx_hbm, tableT, gout,
             idx_v, kept_u, kept_i, staging, sidx,
             *bufs_sems):
    bufs = bufs_sems[:NBUF]
    sems = bufs_sems[NBUF:2 * NBUF]
    sem_s = bufs_sems[2 * NBUF]
    ml = idx_v

    wid = lax.axis_index("s") * NC + lax.axis_index("c")
    wlo = wid * WPW

    pltpu.sync_copy(idx_hbm, idx_v)

    lane = lax.iota(jnp.int32, 16)

    def fbody(j, kept):
        v = idx_v[pl.ds(j * 16, 16)]
        w = v >> 7
        pos = j * 16 + lane
        m = (w >= wlo) & (w < wlo + WPW)
        plsc.store_compressed(kept_u.at[pl.ds(kept, 16)], v, mask=m)
        plsc.store_compressed(kept_i.at[pl.ds(kept, 16)], pos, mask=m)
        return kept + jnp.sum(m.astype(jnp.int32))

    kept_n = lax.fori_loop(0, B // 16, fbody, 0)
    kchunks = (kept_n + 15) >> 4

    for t in range(SROWS // 16):
        sidx[0, pl.ds(t * 16, 16)] = jnp.full((16,), -1, jnp.int32)

    fv = [lane + 16 * t for t in range(4)]

    def fetch(w_abs, buf, sem):
        off = pl.multiple_of(jnp.minimum(w_abs, NWIN - 1) * 128, 128)
        pltpu.async_copy(tableT.at[:, pl.ds(off, 128)], buf, sem)

    def bwait(buf, sem):
        pltpu.make_async_copy(tableT.at[:, pl.ds(0, 128)], buf, sem).wait()

    def flush():
        pltpu.async_copy(
            staging,
            gout.at[plsc.Indices(sidx.at[0], ignored_value=-1)],
            sem_s).wait()
        for t in range(SROWS // 16):
            sidx[0, pl.ds(t * 16, 16)] = jnp.full((16,), -1, jnp.int32)

    def process(buf, w_abs, sc_in):
        def mbody(j, mc):
            u = kept_u[pl.ds(j * 16, 16)]
            pos = j * 16 + lane
            m = ((u >> 7) == w_abs) & (pos < kept_n)
            i = kept_i[pl.ds(j * 16, 16)]
            packed = i | ((u & 127) << 14)
            plsc.store_compressed(ml.at[pl.ds(mc, 16)], packed, mask=m)
            return mc + jnp.sum(m.astype(jnp.int32))

        mcount = lax.fori_loop(0, kchunks, mbody, 0)
        mchunks = (mcount + 15) >> 4

        def cbody(jj, sc):
            mvec = ml[pl.ds(jj * 16, 16)]
            cvec = (mvec >> 14) & 127
            ivec = mvec & 16383
            valid = (jj * 16 + lane) < mcount
            srow = sc & (SROWS - 1)
            sidx[0, pl.ds(srow, 16)] = jnp.where(valid, ivec, -1)
            for k2 in range(16):
                cv = jnp.full((16,), 0, jnp.int32) + cvec[k2]
                for t in range(4):
                    vec = plsc.load_gather(buf, [fv[t], cv])
                    staging[srow + k2, pl.ds(t * 16, 16)] = vec
            sc1 = sc + 16

            @pl.when((sc1 & (SROWS - 1)) == 0)
            def _():
                flush()

            return sc1

        return lax.fori_loop(0, mchunks, cbody, sc_in)

    for b in range(NBUF):
        fetch(wlo + b, bufs[b], sems[b])

    def wbody(g, sc):
        for b in range(NBUF):
            w_abs = wlo + NBUF * g + b
            bwait(bufs[b], sems[b])
            sc = process(bufs[b], w_abs, sc)
            fetch(w_abs + NBUF, bufs[b], sems[b])
        return sc

    sc = lax.fori_loop(0, WPW // NBUF, wbody, 0)
    for b in range(NBUF):
        bwait(bufs[b], sems[b])

    @pl.when((sc & (SROWS - 1)) != 0)
    def _():
        flush()


def _p2_body(g_hbm, item_hbm, wb_hbm, out_hbm,
             g_v, item_v, wb_v, out_v, sem):
    wid = lax.axis_index("s") * NC + lax.axis_index("c")
    base = wid * BPW

    copy = pltpu.async_copy(g_hbm.at[pl.ds(base, BPW), :], g_v, sem)
    pltpu.sync_copy(item_hbm.at[pl.ds(base, BPW)], item_v)
    pltpu.sync_copy(wb_hbm, wb_v)
    copy.wait()

    bvec = plsc.load_gather(wb_v, [jnp.full((16,), F, jnp.int32)])
    lane = lax.iota(jnp.int32, 16)

    def group(g, carry):
        row = g * 16 + lane
        acc = jnp.zeros((16,), jnp.float32)
        for f in range(F):
            fidx = jnp.full((16,), f, jnp.int32)
            uf = plsc.load_gather(g_v, [row, fidx])
            itf = plsc.load_gather(item_v, [row, fidx])
            wf = plsc.load_gather(wb_v, [fidx])
            acc = acc + uf * itf * wf
        logit = acc + bvec
        out_v[pl.ds(g * 16, 16)] = 1.0 / (1.0 + jnp.exp(-logit))
        return carry

    lax.fori_loop(0, GROUPS, group, 0)
    pltpu.sync_copy(out_v, out_hbm.at[pl.ds(base, BPW)])


@jax.jit
def _gmf(user_idx_c, item_embedding, emb_table, wb):
    mesh = plsc.VectorSubcoreMesh(core_axis_name="c", subcore_axis_name="s")
    tableT = emb_table.T

    gathered = pl.kernel(
        _p1_body,
        out_type=jax.ShapeDtypeStruct((B, 128), jnp.float32),
        mesh=mesh,
        compiler_params=pltpu.CompilerParams(
            needs_layout_passes=False, use_tc_tiling_on_sc=True),
        scratch_types=[
            pltpu.VMEM((B,), jnp.int32),
            pltpu.VMEM((B,), jnp.int32),
            pltpu.VMEM((B,), jnp.int32),
            pltpu.VMEM((SROWS, 128), jnp.float32),
            pltpu.VMEM((1, SROWS), jnp.int32),
        ] + [pltpu.VMEM((F, 128), jnp.float32) for _ in range(NBUF)]
          + [pltpu.SemaphoreType.DMA for _ in range(NBUF + 1)],
    )(user_idx_c, tableT)

    out = pl.kernel(
        _p2_body,
        out_type=jax.ShapeDtypeStruct((B,), jnp.float32),
        mesh=mesh,
        compiler_params=pltpu.CompilerParams(
            needs_layout_passes=False, use_tc_tiling_on_sc=False),
        scratch_types=[
            pltpu.VMEM((BPW, 128), jnp.float32),
            pltpu.VMEM((BPW, F), jnp.float32),
            pltpu.VMEM((F + 16,), jnp.float32),
            pltpu.VMEM((BPW,), jnp.float32),
            pltpu.SemaphoreType.DMA,
        ],
    )(gathered, item_embedding, wb)
    return out


def kernel(user_idx, item_embedding, emb_table, fc_w, fc_b):
    idx = user_idx.astype(jnp.int32)
    wb = jnp.concatenate(
        [fc_w.reshape(F), jnp.broadcast_to(fc_b.reshape(1), (16,))]
    ).astype(jnp.float32)
    return _gmf(idx, item_embedding, emb_table, wb)


# speedup vs baseline: 2.8830x; 1.6597x over previous
import functools
import jax
import jax.numpy as jnp
from jax import lax
from jax.experimental import pallas as pl
from jax.experimental.pallas import tpu as pltpu, tpu_sc as plsc

USERS = 1000000
F = 64
B = 16384
NC = 2
NS = 16
NW = NC * NS
BPW = B // NW
NWIN = (USERS + 127) // 128
NBUF = 4
WPW = 248
GROUPS = BPW // 16
SROWS = 64
PPW = WPW // 2
CLAMP = (NWIN - 2) * 128


def _p1_body(idx_hbm, tableT, gout,
             idx_v, kept_u, kept_i, staging, sidx,
             *bufs_sems):
    bufs = bufs_sems[:NBUF]
    sems = bufs_sems[NBUF:2 * NBUF]
    sem_s = bufs_sems[2 * NBUF]
    ml = idx_v

    wid = lax.axis_index("s") * NC + lax.axis_index("c")
    wlo = wid * WPW

    pltpu.sync_copy(idx_hbm, idx_v)

    lane = lax.iota(jnp.int32, 16)

    def fbody(j, kept):
        v = idx_v[pl.ds(j * 16, 16)]
        w = v >> 7
        pos = j * 16 + lane
        m = (w >= wlo) & (w < wlo + WPW)
        ks = kept[0]
        plsc.store_compressed(kept_u.at[pl.ds(ks, 16)], v, mask=m)
        plsc.store_compressed(kept_i.at[pl.ds(ks, 16)], pos, mask=m)
        return kept + plsc.all_reduce_population_count(m)

    kept_n = lax.fori_loop(0, B // 16, fbody,
                           jnp.zeros((16,), jnp.int32))[0]
    kchunks = (kept_n + 15) >> 4

    for t in range(SROWS // 16):
        sidx[0, pl.ds(t * 16, 16)] = jnp.full((16,), -1, jnp.int32)

    fv = [lane + 16 * t for t in range(4)]

    def fetch(p_abs, buf, sem):
        off = pl.multiple_of(jnp.minimum(p_abs * 256, CLAMP), 128)
        pltpu.async_copy(tableT.at[:, pl.ds(off, 256)], buf, sem)

    def bwait(buf, sem):
        pltpu.make_async_copy(tableT.at[:, pl.ds(0, 256)], buf, sem).wait()

    def flush():
        pltpu.async_copy(
            staging,
            gout.at[plsc.Indices(sidx.at[0], ignored_value=-1)],
            sem_s).wait()
        for t in range(SROWS // 16):
            sidx[0, pl.ds(t * 16, 16)] = jnp.full((16,), -1, jnp.int32)

    def process(buf, p_abs, sc_in):
        off = jnp.minimum(p_abs * 256, CLAMP)

        def mbody(j, mc):
            u = kept_u[pl.ds(j * 16, 16)]
            pos = j * 16 + lane
            m = (u >= off) & (u < off + 256) & (pos < kept_n)
            i = kept_i[pl.ds(j * 16, 16)]
            packed = i | ((u - off) << 14)
            plsc.store_compressed(ml.at[pl.ds(mc[0], 16)], packed, mask=m)
            return mc + plsc.all_reduce_population_count(m)

        mcount = lax.fori_loop(0, kchunks, mbody,
                               jnp.zeros((16,), jnp.int32))[0]
        mchunks = (mcount + 15) >> 4

        def cbody(jj, sc):
            mvec = ml[pl.ds(jj * 16, 16)]
            cvec = (mvec >> 14) & 255
            ivec = mvec & 16383
            valid = (jj * 16 + lane) < mcount
            srow = sc & (SROWS - 1)
            sidx[0, pl.ds(srow, 16)] = jnp.where(valid, ivec, -1)
            for k2 in range(16):
                cv = jnp.full((16,), 0, jnp.int32) + cvec[k2]
                for t in range(4):
                    vec = plsc.load_gather(buf, [fv[t], cv])
                    staging[srow + k2, pl.ds(t * 16, 16)] = vec
            sc1 = sc + 16

            @pl.when((sc1 & (SROWS - 1)) == 0)
            def _():
                flush()

            return sc1

        return lax.fori_loop(0, mchunks, cbody, sc_in)

    plo = wlo >> 1
    for b in range(NBUF):
        fetch(plo + b, bufs[b], sems[b])

    def wbody(g, sc):
        for b in range(NBUF):
            p_abs = plo + NBUF * g + b
            bwait(bufs[b], sems[b])
            sc = process(bufs[b], p_abs, sc)
            fetch(p_abs + NBUF, bufs[b], sems[b])
        return sc

    sc = lax.fori_loop(0, PPW // NBUF, wbody, 0)
    for b in range(NBUF):
        bwait(bufs[b], sems[b])

    @pl.when((sc & (SROWS - 1)) != 0)
    def _():
        flush()


def _p2_body(g_hbm, item_hbm, wb_hbm, out_hbm,
             g_v, item_v, wb_v, out_v, sem):
    wid = lax.axis_index("s") * NC + lax.axis_index("c")
    base = wid * BPW

    copy = pltpu.async_copy(g_hbm.at[pl.ds(base, BPW), :], g_v, sem)
    pltpu.sync_copy(item_hbm.at[pl.ds(base, BPW)], item_v)
    pltpu.sync_copy(wb_hbm, wb_v)
    copy.wait()

    bvec = plsc.load_gather(wb_v, [jnp.full((16,), F, jnp.int32)])
    lane = lax.iota(jnp.int32, 16)

    def group(g, carry):
        row = g * 16 + lane
        acc = jnp.zeros((16,), jnp.float32)
        for f in range(F):
            fidx = jnp.full((16,), f, jnp.int32)
            uf = plsc.load_gather(g_v, [row, fidx])
            itf = plsc.load_gather(item_v, [row, fidx])
            wf = plsc.load_gather(wb_v, [fidx])
            acc = acc + uf * itf * wf
        logit = acc + bvec
        out_v[pl.ds(g * 16, 16)] = 1.0 / (1.0 + jnp.exp(-logit))
        return carry

    lax.fori_loop(0, GROUPS, group, 0)
    pltpu.sync_copy(out_v, out_hbm.at[pl.ds(base, BPW)])


@jax.jit
def _gmf(user_idx_c, item_embedding, emb_table, wb):
    mesh = plsc.VectorSubcoreMesh(core_axis_name="c", subcore_axis_name="s")
    tableT = emb_table.T

    gathered = pl.kernel(
        _p1_body,
        out_type=jax.ShapeDtypeStruct((B, 128), jnp.float32),
        mesh=mesh,
        compiler_params=pltpu.CompilerParams(
            needs_layout_passes=False, use_tc_tiling_on_sc=True),
        scratch_types=[
            pltpu.VMEM((B,), jnp.int32),
            pltpu.VMEM((B,), jnp.int32),
            pltpu.VMEM((B,), jnp.int32),
            pltpu.VMEM((SROWS, 128), jnp.float32),
            pltpu.VMEM((1, SROWS), jnp.int32),
        ] + [pltpu.VMEM((F, 256), jnp.float32) for _ in range(NBUF)]
          + [pltpu.SemaphoreType.DMA for _ in range(NBUF + 1)],
    )(user_idx_c, tableT)

    out = pl.kernel(
        _p2_body,
        out_type=jax.ShapeDtypeStruct((B,), jnp.float32),
        mesh=mesh,
        compiler_params=pltpu.CompilerParams(
            needs_layout_passes=False, use_tc_tiling_on_sc=False),
        scratch_types=[
            pltpu.VMEM((BPW, 128), jnp.float32),
            pltpu.VMEM((BPW, F), jnp.float32),
            pltpu.VMEM((F + 16,), jnp.float32),
            pltpu.VMEM((BPW,), jnp.float32),
            pltpu.SemaphoreType.DMA,
        ],
    )(gathered, item_embedding, wb)
    return out


def kernel(user_idx, item_embedding, emb_table, fc_w, fc_b):
    idx = user_idx.astype(jnp.int32)
    wb = jnp.concatenate(
        [fc_w.reshape(F), jnp.broadcast_to(fc_b.reshape(1), (16,))]
    ).astype(jnp.float32)
    return _gmf(idx, item_embedding, emb_table, wb)


# speedup vs baseline: 2.9582x; 1.0261x over previous
import functools
import jax
import jax.numpy as jnp
from jax import lax
from jax.experimental import pallas as pl
from jax.experimental.pallas import tpu as pltpu, tpu_sc as plsc

USERS = 1000000
F = 64
B = 16384
NC = 2
NS = 16
NW = NC * NS
BPW = B // NW
NWIN = (USERS + 127) // 128
NBUF = 4
WPW = 248
GROUPS = BPW // 16
SROWS = 64
PPW = WPW // 2
CLAMP = (NWIN - 2) * 128


def _p1_body(idx_hbm, tableT, gout,
             idx_v, kept_u, kept_i, staging, sidx,
             *bufs_sems):
    bufs = bufs_sems[:NBUF]
    sems = bufs_sems[NBUF:2 * NBUF]
    sem_s = bufs_sems[2 * NBUF]
    ml = idx_v

    wid = lax.axis_index("s") * NC + lax.axis_index("c")
    wlo = wid * WPW

    pltpu.sync_copy(idx_hbm, idx_v)

    lane = lax.iota(jnp.int32, 16)

    def fbody(j, kept):
        v = idx_v[pl.ds(j * 16, 16)]
        w = v >> 7
        pos = j * 16 + lane
        m = (w >= wlo) & (w < wlo + WPW)
        ks = kept[0]
        plsc.store_compressed(kept_u.at[pl.ds(ks, 16)], v, mask=m)
        plsc.store_compressed(kept_i.at[pl.ds(ks, 16)], pos, mask=m)
        return kept + plsc.all_reduce_population_count(m)

    kept_n = lax.fori_loop(0, B // 16, fbody,
                           jnp.zeros((16,), jnp.int32))[0]
    kchunks = (kept_n + 15) >> 4

    for t in range(SROWS // 16):
        sidx[0, pl.ds(t * 16, 16)] = jnp.full((16,), -1, jnp.int32)

    fv = [lane + 16 * t for t in range(4)]

    def fetch(p_abs, buf, sem):
        off = pl.multiple_of(jnp.minimum(p_abs * 256, CLAMP), 128)
        pltpu.async_copy(tableT.at[:, pl.ds(off, 256)], buf, sem)

    def bwait(buf, sem):
        pltpu.make_async_copy(tableT.at[:, pl.ds(0, 256)], buf, sem).wait()

    def flush():
        pltpu.async_copy(
            staging,
            gout.at[plsc.Indices(sidx.at[0], ignored_value=-1)],
            sem_s).wait()
        for t in range(SROWS // 16):
            sidx[0, pl.ds(t * 16, 16)] = jnp.full((16,), -1, jnp.int32)

    def process(buf, p_abs, sc_in):
        off = jnp.minimum(p_abs * 256, CLAMP)

        def mbody(j, mc):
            u = kept_u[pl.ds(j * 16, 16)]
            pos = j * 16 + lane
            m = (u >= off) & (u < off + 256) & (pos < kept_n)
            i = kept_i[pl.ds(j * 16, 16)]
            packed = i | ((u - off) << 14)
            plsc.store_compressed(ml.at[pl.ds(mc[0], 16)], packed, mask=m)
            return mc + plsc.all_reduce_population_count(m)

        mcount = lax.fori_loop(0, kchunks, mbody,
                               jnp.zeros((16,), jnp.int32))[0]
        mchunks = (mcount + 15) >> 4

        def cbody(jj, sc):
            mvec = ml[pl.ds(jj * 16, 16)]
            cvec = (mvec >> 14) & 255
            ivec = mvec & 16383
            valid = (jj * 16 + lane) < mcount
            srow = sc & (SROWS - 1)
            sidx[0, pl.ds(srow, 16)] = jnp.where(valid, ivec, -1)
            for k2 in range(16):
                cv = jnp.full((16,), 0, jnp.int32) + cvec[k2]
                for t in range(4):
                    vec = plsc.load_gather(buf, [fv[t], cv])
                    staging[srow + k2, pl.ds(t * 16, 16)] = vec
            sc1 = sc + 16

            @pl.when((sc1 & (SROWS - 1)) == 0)
            def _():
                flush()

            return sc1

        return lax.fori_loop(0, mchunks, cbody, sc_in)

    plo = wlo >> 1
    for b in range(NBUF):
        fetch(plo + b, bufs[b], sems[b])

    def wbody(g, sc):
        for b in range(NBUF):
            p_abs = plo + NBUF * g + b
            bwait(bufs[b], sems[b])
            sc = process(bufs[b], p_abs, sc)
            fetch(p_abs + NBUF, bufs[b], sems[b])
        return sc

    sc = lax.fori_loop(0, PPW // NBUF, wbody, 0)
    for b in range(NBUF):
        bwait(bufs[b], sems[b])

    @pl.when((sc & (SROWS - 1)) != 0)
    def _():
        flush()


def _p2_body(g_hbm, itemT_hbm, wb_hbm, out_hbm,
             g_v, item_v, wb_v, out_v, sem):
    wid = lax.axis_index("s") * NC + lax.axis_index("c")
    base = wid * BPW

    copy = pltpu.async_copy(g_hbm.at[pl.ds(base, BPW), :], g_v, sem)
    pltpu.sync_copy(itemT_hbm.at[:, pl.ds(base, BPW)], item_v)
    pltpu.sync_copy(wb_hbm, wb_v)
    copy.wait()

    zero = jnp.full((16,), 0, jnp.int32)
    bvec = plsc.load_gather(wb_v, [zero, zero + F])
    lane = lax.iota(jnp.int32, 16)

    def group(g, carry):
        row = g * 16 + lane
        accs = [jnp.zeros((16,), jnp.float32) for _ in range(4)]
        for f in range(F):
            fidx = zero + f
            uf = plsc.load_gather(g_v, [row, fidx])
            itf = plsc.load_gather(item_v, [fidx, row])
            wf = plsc.load_gather(wb_v, [zero, fidx])
            accs[f % 4] = accs[f % 4] + uf * itf * wf
        logit = (accs[0] + accs[1]) + (accs[2] + accs[3]) + bvec
        out_v[0, pl.ds(g * 16, 16)] = 1.0 / (1.0 + jnp.exp(-logit))
        return carry

    lax.fori_loop(0, GROUPS, group, 0)
    pltpu.sync_copy(out_v, out_hbm.at[wid])


@jax.jit
def _gmf(user_idx_c, item_embedding, emb_table, wb):
    mesh = plsc.VectorSubcoreMesh(core_axis_name="c", subcore_axis_name="s")
    tableT = emb_table.T

    gathered = pl.kernel(
        _p1_body,
        out_type=jax.ShapeDtypeStruct((B, 128), jnp.float32),
        mesh=mesh,
        compiler_params=pltpu.CompilerParams(
            needs_layout_passes=False, use_tc_tiling_on_sc=True),
        scratch_types=[
            pltpu.VMEM((B,), jnp.int32),
            pltpu.VMEM((B,), jnp.int32),
            pltpu.VMEM((B,), jnp.int32),
            pltpu.VMEM((SROWS, 128), jnp.float32),
            pltpu.VMEM((1, SROWS), jnp.int32),
        ] + [pltpu.VMEM((F, 256), jnp.float32) for _ in range(NBUF)]
          + [pltpu.SemaphoreType.DMA for _ in range(NBUF + 1)],
    )(user_idx_c, tableT)

    itemT = item_embedding.T
    out3 = pl.kernel(
        _p2_body,
        out_type=jax.ShapeDtypeStruct((NW, 1, BPW), jnp.float32),
        mesh=mesh,
        compiler_params=pltpu.CompilerParams(
            needs_layout_passes=False, use_tc_tiling_on_sc=True),
        scratch_types=[
            pltpu.VMEM((BPW, 128), jnp.float32),
            pltpu.VMEM((F, BPW), jnp.float32),
            pltpu.VMEM((1, 128), jnp.float32),
            pltpu.VMEM((1, BPW), jnp.float32),
            pltpu.SemaphoreType.DMA,
        ],
    )(gathered, itemT, wb)
    return out3.reshape(B)


def kernel(user_idx, item_embedding, emb_table, fc_w, fc_b):
    idx = user_idx.astype(jnp.int32)
    wb = jnp.concatenate(
        [fc_w.reshape(F), jnp.broadcast_to(fc_b.reshape(1), (16,)),
         jnp.zeros((48,), jnp.float32)]
    ).astype(jnp.float32).reshape(1, 128)
    return _gmf(idx, item_embedding, emb_table, wb)


# speedup vs baseline: 3.6053x; 1.2188x over previous
import functools
import jax
import jax.numpy as jnp
from jax import lax
from jax.experimental import pallas as pl
from jax.experimental.pallas import tpu as pltpu, tpu_sc as plsc

USERS = 1000000
F = 64
B = 16384
NC = 2
NS = 16
NW = NC * NS
BPW = B // NW
NWIN = (USERS + 127) // 128
NBUF = 3
WPW = 252
GROUPS = BPW // 16
SROWS = 32
PPW = WPW // 3
CLAMP = (NWIN - 3) * 128


def _p1_body(idx_hbm, tableT, gout,
             idx_v, kept_u, kept_i, staging, sidx,
             *bufs_sems):
    bufs = bufs_sems[:NBUF]
    sems = bufs_sems[NBUF:2 * NBUF]
    sem_s = bufs_sems[2 * NBUF]
    ml = idx_v

    wid = lax.axis_index("s") * NC + lax.axis_index("c")
    wlo = wid * WPW

    pltpu.sync_copy(idx_hbm, idx_v)

    lane = lax.iota(jnp.int32, 16)

    def fbody(j, kept):
        v = idx_v[pl.ds(j * 16, 16)]
        w = v >> 7
        pos = j * 16 + lane
        m = (w >= wlo) & (w < wlo + WPW)
        ks = kept[0]
        plsc.store_compressed(kept_u.at[pl.ds(ks, 16)], v, mask=m)
        plsc.store_compressed(kept_i.at[pl.ds(ks, 16)], pos, mask=m)
        return kept + plsc.all_reduce_population_count(m)

    kept_n = lax.fori_loop(0, B // 16, fbody,
                           jnp.zeros((16,), jnp.int32))[0]
    kchunks = (kept_n + 15) >> 4

    for t in range(SROWS // 16):
        sidx[0, pl.ds(t * 16, 16)] = jnp.full((16,), -1, jnp.int32)

    fv = [lane + 16 * t for t in range(4)]

    def fetch(p_abs, buf, sem):
        off = pl.multiple_of(jnp.minimum(p_abs * 384, CLAMP), 128)
        pltpu.async_copy(tableT.at[:, pl.ds(off, 384)], buf, sem)

    def bwait(buf, sem):
        pltpu.make_async_copy(tableT.at[:, pl.ds(0, 384)], buf, sem).wait()

    def flush():
        pltpu.async_copy(
            staging,
            gout.at[plsc.Indices(sidx.at[0], ignored_value=-1)],
            sem_s).wait()
        for t in range(SROWS // 16):
            sidx[0, pl.ds(t * 16, 16)] = jnp.full((16,), -1, jnp.int32)

    def process(buf, p_abs, sc_in):
        off = jnp.minimum(p_abs * 384, CLAMP)

        def mbody(j, mc):
            u = kept_u[pl.ds(j * 16, 16)]
            pos = j * 16 + lane
            m = (u >= off) & (u < off + 384) & (pos < kept_n)
            i = kept_i[pl.ds(j * 16, 16)]
            packed = i | ((u - off) << 14)
            plsc.store_compressed(ml.at[pl.ds(mc[0], 16)], packed, mask=m)
            return mc + plsc.all_reduce_population_count(m)

        mcount = lax.fori_loop(0, kchunks, mbody,
                               jnp.zeros((16,), jnp.int32))[0]
        mchunks = (mcount + 15) >> 4

        def cbody(jj, sc):
            mvec = ml[pl.ds(jj * 16, 16)]
            cvec = (mvec >> 14) & 511
            ivec = mvec & 16383
            valid = (jj * 16 + lane) < mcount
            srow = sc & (SROWS - 1)
            sidx[0, pl.ds(srow, 16)] = jnp.where(valid, ivec, -1)
            for k2 in range(16):
                cv = jnp.full((16,), 0, jnp.int32) + cvec[k2]
                for t in range(4):
                    vec = plsc.load_gather(buf, [fv[t], cv])
                    staging[srow + k2, pl.ds(t * 16, 16)] = vec
            sc1 = sc + 16

            @pl.when((sc1 & (SROWS - 1)) == 0)
            def _():
                flush()

            return sc1

        return lax.fori_loop(0, mchunks, cbody, sc_in)

    plo = wid * PPW
    for b in range(NBUF):
        fetch(plo + b, bufs[b], sems[b])

    def wbody(g, sc):
        for b in range(NBUF):
            p_abs = plo + NBUF * g + b
            bwait(bufs[b], sems[b])
            sc = process(bufs[b], p_abs, sc)
            fetch(p_abs + NBUF, bufs[b], sems[b])
        return sc

    sc = lax.fori_loop(0, PPW // NBUF, wbody, 0)
    for b in range(NBUF):
        bwait(bufs[b], sems[b])

    @pl.when((sc & (SROWS - 1)) != 0)
    def _():
        flush()


def _p2_body(g_hbm, itemT_hbm, wb_hbm, out_hbm,
             g_v, item_v, wb_v, out_v, sem):
    wid = lax.axis_index("s") * NC + lax.axis_index("c")
    base = wid * BPW

    copy = pltpu.async_copy(g_hbm.at[pl.ds(base, BPW), :], g_v, sem)
    pltpu.sync_copy(itemT_hbm.at[:, pl.ds(base, BPW)], item_v)
    pltpu.sync_copy(wb_hbm, wb_v)
    copy.wait()

    zero = jnp.full((16,), 0, jnp.int32)
    bvec = plsc.load_gather(wb_v, [zero, zero + F])
    lane = lax.iota(jnp.int32, 16)

    def group(g, carry):
        row = g * 16 + lane
        accs = [jnp.zeros((16,), jnp.float32) for _ in range(4)]
        for f in range(F):
            fidx = zero + f
            uf = plsc.load_gather(g_v, [row, fidx])
            itf = plsc.load_gather(item_v, [fidx, row])
            wf = plsc.load_gather(wb_v, [zero, fidx])
            accs[f % 4] = accs[f % 4] + uf * itf * wf
        logit = (accs[0] + accs[1]) + (accs[2] + accs[3]) + bvec
        out_v[0, pl.ds(g * 16, 16)] = 1.0 / (1.0 + jnp.exp(-logit))
        return carry

    lax.fori_loop(0, GROUPS, group, 0)
    pltpu.sync_copy(out_v, out_hbm.at[wid])


@jax.jit
def _gmf(user_idx_c, item_embedding, emb_table, wb):
    mesh = plsc.VectorSubcoreMesh(core_axis_name="c", subcore_axis_name="s")
    tableT = emb_table.T

    gathered = pl.kernel(
        _p1_body,
        out_type=jax.ShapeDtypeStruct((B, 128), jnp.float32),
        mesh=mesh,
        compiler_params=pltpu.CompilerParams(
            needs_layout_passes=False, use_tc_tiling_on_sc=True),
        scratch_types=[
            pltpu.VMEM((B,), jnp.int32),
            pltpu.VMEM((B,), jnp.int32),
            pltpu.VMEM((B,), jnp.int32),
            pltpu.VMEM((SROWS, 128), jnp.float32),
            pltpu.VMEM((1, SROWS), jnp.int32),
        ] + [pltpu.VMEM((F, 384), jnp.float32) for _ in range(NBUF)]
          + [pltpu.SemaphoreType.DMA for _ in range(NBUF + 1)],
    )(user_idx_c, tableT)

    itemT = item_embedding.T
    out3 = pl.kernel(
        _p2_body,
        out_type=jax.ShapeDtypeStruct((NW, 1, BPW), jnp.float32),
        mesh=mesh,
        compiler_params=pltpu.CompilerParams(
            needs_layout_passes=False, use_tc_tiling_on_sc=True),
        scratch_types=[
            pltpu.VMEM((BPW, 128), jnp.float32),
            pltpu.VMEM((F, BPW), jnp.float32),
            pltpu.VMEM((1, 128), jnp.float32),
            pltpu.VMEM((1, BPW), jnp.float32),
            pltpu.SemaphoreType.DMA,
        ],
    )(gathered, itemT, wb)
    return out3.reshape(B)


def kernel(user_idx, item_embedding, emb_table, fc_w, fc_b):
    idx = user_idx.astype(jnp.int32)
    wb = jnp.concatenate(
        [fc_w.reshape(F), jnp.broadcast_to(fc_b.reshape(1), (16,)),
         jnp.zeros((48,), jnp.float32)]
    ).astype(jnp.float32).reshape(1, 128)
    return _gmf(idx, item_embedding, emb_table, wb)
